# baseline (speedup 1.0000x reference)
import functools
import math

import jax
import jax.numpy as jnp
from jax import lax
from jax.experimental import pallas as pl
from jax.experimental.pallas import tpu as pltpu
from jax.experimental.pallas import tpu_sc as plsc

DIM = 256
N = 256
SIGMA_D = 0.2
SIGMA_A = 15.0
ANGLE_K = 3
FACTOR_A = 180.0 / (SIGMA_A * math.pi)

RT = 512
TROWS = 12288
BINS_A = 8192
A_STEP = 12.0 / (BINS_A - 1)
BI = 8

NC, NS = 2, 16
NW = NC * NS
RPW = (N * N) // NW
CH = 64
NCH = RPW // CH


def _tables_body(wsd_ref, wcd_ref, bd_ref, wsa_ref, wca_ref, ba_ref,
                 div_ref, td_ref, ta_ref):
    base = pl.program_id(0) * RT
    r = (jax.lax.broadcasted_iota(jnp.int32, (RT, 1), 0)
         + base).astype(jnp.float32)
    div = div_ref[0:1, :]
    omd = (jnp.sqrt(r) / jnp.float32(SIGMA_D)) * div
    td_ref[...] = (jnp.dot(jnp.sin(omd), wsd_ref[...],
                           preferred_element_type=jnp.float32)
                   + jnp.dot(jnp.cos(omd), wcd_ref[...],
                             preferred_element_type=jnp.float32)
                   + bd_ref[...]).astype(jnp.bfloat16)
    oma = (r * jnp.float32(A_STEP)) * div
    ta_ref[...] = (jnp.dot(jnp.sin(oma), wsa_ref[...],
                           preferred_element_type=jnp.float32)
                   + jnp.dot(jnp.cos(oma), wca_ref[...],
                             preferred_element_type=jnp.float32)
                   + ba_ref[...]).astype(jnp.bfloat16)


def _bins_body(pft_ref, pi_ref, bd_ref, ba_ref):
    pft = pft_ref[...]
    xj = pft[0:1, :]
    yj = pft[1:2, :]
    zj = pft[2:3, :]
    xi = pi_ref[:, 0:1]
    yi = pi_ref[:, 1:2]
    zi = pi_ref[:, 2:3]

    ax = xj - xi
    ay = yj - yi
    az = zj - zi
    s2 = ax * ax + ay * ay + az * az

    bd_ref[...] = s2.astype(jnp.int32)

    iota = jax.lax.broadcasted_iota(jnp.int32, (BI, N), 1)
    s2m = s2
    knn = []
    for t in range(ANGLE_K + 1):
        mn = jnp.min(s2m, axis=1, keepdims=True)
        idx = jnp.min(jnp.where(s2m == mn, iota, N), axis=1, keepdims=True)
        if t > 0:
            knn.append(idx)
        s2m = jnp.where(iota == idx, jnp.float32(jnp.inf), s2m)

    scale = jnp.float32(FACTOR_A * (1.0 / A_STEP))
    for t in range(ANGLE_K):
        onehot = (iota == knn[t]).astype(jnp.float32)
        rx = jnp.sum(onehot * xj, axis=1, keepdims=True) - xi
        ry = jnp.sum(onehot * yj, axis=1, keepdims=True) - yi
        rz = jnp.sum(onehot * zj, axis=1, keepdims=True) - zi
        cosv = rx * ax + ry * ay + rz * az
        cx = ry * az - rz * ay
        cy = rz * ax - rx * az
        cz = rx * ay - ry * ax
        sinv = jnp.sqrt(cx * cx + cy * cy + cz * cz)
        ang = jnp.arctan2(sinv, cosv)
        ba_ref[t] = jnp.round(ang * scale).astype(jnp.int32)


def _sc_body(td_hbm, ta_hbm, bd_hbm, ba0_hbm, ba1_hbm, ba2_hbm, out_hbm,
             idxd_v, idxa0_v, idxa1_v, idxa2_v,
             rd0, a00, a10, a20, ob0, rd1, a01, a11, a21, ob1, gs0, gs1):
    wid = lax.axis_index("s") * NC + lax.axis_index("c")
    base = wid * RPW
    pltpu.sync_copy(bd_hbm.at[pl.ds(base, RPW)], idxd_v)
    pltpu.sync_copy(ba0_hbm.at[pl.ds(base, RPW)], idxa0_v)
    pltpu.sync_copy(ba1_hbm.at[pl.ds(base, RPW)], idxa1_v)
    pltpu.sync_copy(ba2_hbm.at[pl.ds(base, RPW)], idxa2_v)

    bufs = ((rd0, a00, a10, a20, ob0, gs0), (rd1, a01, a11, a21, ob1, gs1))

    def fire(g, s):
        off = g * CH
        rd, a0, a1, a2, ob, gs = bufs[s]
        pltpu.async_copy(td_hbm.at[idxd_v.at[pl.ds(off, CH)]], rd, gs)
        pltpu.async_copy(ta_hbm.at[idxa0_v.at[pl.ds(off, CH)]], a0, gs)
        pltpu.async_copy(ta_hbm.at[idxa1_v.at[pl.ds(off, CH)]], a1, gs)
        pltpu.async_copy(ta_hbm.at[idxa2_v.at[pl.ds(off, CH)]], a2, gs)

    def drain(s):
        rd, a0, a1, a2, ob, gs = bufs[s]
        for dst in (rd, a0, a1, a2):
            pltpu.make_async_copy(td_hbm.at[pl.ds(0, CH)], dst, gs).wait()

    HMASK = jnp.int32(-65536)

    def compute_store(g, s):
        off = g * CH
        rd, a0, a1, a2, ob, gs = bufs[s]

        def unpk(w):
            return (jax.lax.bitcast_convert_type(w << 16, jnp.float32),
                    jax.lax.bitcast_convert_type(w & HMASK, jnp.float32))

        def rowbody(rr, c2):
            for c in range(DIM // 32):
                sl = pl.ds(c * 16, 16)
                a0e, a0o = unpk(a0[rr, sl])
                a1e, a1o = unpk(a1[rr, sl])
                a2e, a2o = unpk(a2[rr, sl])
                t_ev, t_od = unpk(rd[rr, sl])
                m_ev = jnp.maximum(jnp.maximum(a0e, a1e), a2e)
                m_od = jnp.maximum(jnp.maximum(a0o, a1o), a2o)
                ob[rr, pl.ds(c * 32, 16)] = t_ev + m_ev
                ob[rr, pl.ds(c * 32 + 16, 16)] = t_od + m_od
            return c2
        lax.fori_loop(0, CH, rowbody, 0)
        pltpu.sync_copy(ob, out_hbm.at[pl.ds(base + off, CH)])

    fire(0, 0)

    def outer(t, carry):
        g0 = t * 2
        fire(g0 + 1, 1)
        drain(0)
        compute_store(g0, 0)

        @pl.when(g0 + 2 < NCH)
        def _():
            fire(g0 + 2, 0)
        drain(1)
        compute_store(g0 + 1, 1)
        return carry

    lax.fori_loop(0, NCH // 2, outer, 0)


def _swizzle_perm():
    p = []
    for g in range(DIM // 32):
        for i in range(16):
            p.append(32 * g + i)
            p.append(32 * g + 16 + i)
    return p


_PERM = tuple(_swizzle_perm())


@jax.jit
def kernel(points, Wd, bd, Wa, ba):
    pft = points.reshape(N, 3).T.astype(jnp.float32)
    perm = jnp.asarray(_PERM, dtype=jnp.int32)
    WdT = Wd.T[:, perm]
    WaT = Wa.T[:, perm]
    bd = bd[perm]
    ba = ba[perm]
    wsd, wcd = WdT[0::2], WdT[1::2]
    wsa, wca = WaT[0::2], WaT[1::2]
    div = jnp.exp(jnp.arange(0, DIM, 2, dtype=jnp.float32)
                  * (-math.log(10000.0) / DIM)).reshape(1, DIM // 2)

    full = lambda i: (0, 0)
    wspec = pl.BlockSpec((DIM // 2, DIM), full)
    bspec = pl.BlockSpec((1, DIM), full)

    td, ta = pl.pallas_call(
        _tables_body,
        grid=(TROWS // RT,),
        in_specs=[wspec, wspec, bspec, wspec, wspec, bspec,
                  pl.BlockSpec((1, DIM // 2), full)],
        out_specs=[pl.BlockSpec((RT, DIM), lambda i: (i, 0)),
                   pl.BlockSpec((RT, DIM), lambda i: (i, 0))],
        out_shape=[jax.ShapeDtypeStruct((TROWS, DIM), jnp.bfloat16),
                   jax.ShapeDtypeStruct((TROWS, DIM), jnp.bfloat16)],
    )(wsd, wcd, bd.reshape(1, DIM), wsa, wca, ba.reshape(1, DIM), div)
    tdw = jax.lax.bitcast_convert_type(
        td.reshape(TROWS, DIM // 2, 2), jnp.int32)
    taw = jax.lax.bitcast_convert_type(
        ta.reshape(TROWS, DIM // 2, 2), jnp.int32)

    bd_idx, ba_idx = pl.pallas_call(
        _bins_body,
        grid=(N // BI,),
        in_specs=[pl.BlockSpec((3, N), full),
                  pl.BlockSpec((BI, 3), lambda i: (i, 0))],
        out_specs=[pl.BlockSpec((BI, N), lambda i: (i, 0)),
                   pl.BlockSpec((ANGLE_K, BI, N), lambda i: (0, i, 0))],
        out_shape=[jax.ShapeDtypeStruct((N, N), jnp.int32),
                   jax.ShapeDtypeStruct((ANGLE_K, N, N), jnp.int32)],
    )(pft, pft.T)

    sc = pl.kernel(
        _sc_body,
        out_type=jax.ShapeDtypeStruct((N * N, DIM), jnp.float32),
        mesh=plsc.VectorSubcoreMesh(core_axis_name="c", subcore_axis_name="s"),
        scratch_types=(
            [pltpu.VMEM((RPW,), jnp.int32)] * 4
            + ([pltpu.VMEM((CH, DIM // 2), jnp.int32)] * 4
               + [pltpu.VMEM((CH, DIM), jnp.float32)]) * 2
            + [pltpu.SemaphoreType.DMA, pltpu.SemaphoreType.DMA]
        ),
    )
    ba_flat = ba_idx.reshape(ANGLE_K, N * N)
    out = sc(tdw, taw, bd_idx.reshape(N * N),
             ba_flat[0], ba_flat[1], ba_flat[2])
    return out.reshape(1, N, N, DIM)


# speedup vs baseline: 1.0165x; 1.0165x over previous
import functools
import math

import jax
import jax.numpy as jnp
from jax import lax
from jax.experimental import pallas as pl
from jax.experimental.pallas import tpu as pltpu
from jax.experimental.pallas import tpu_sc as plsc

DIM = 256
N = 256
SIGMA_D = 0.2
SIGMA_A = 15.0
ANGLE_K = 3
FACTOR_A = 180.0 / (SIGMA_A * math.pi)

RT = 512
TROWS = 12288
BINS_A = 8192
A_STEP = 12.0 / (BINS_A - 1)
BI = 8

NC, NS = 2, 16
NW = NC * NS
RPW = (N * N) // NW
CH = 64
NCH = RPW // CH


def _tables_body(wsd_ref, wcd_ref, bd_ref, wsa_ref, wca_ref, ba_ref,
                 div_ref, td_ref, ta_ref):
    base = pl.program_id(0) * RT
    r = (jax.lax.broadcasted_iota(jnp.int32, (RT, 1), 0)
         + base).astype(jnp.float32)
    div = div_ref[0:1, :]
    omd = (jnp.sqrt(r) / jnp.float32(SIGMA_D)) * div
    td_ref[...] = (jnp.dot(jnp.sin(omd), wsd_ref[...],
                           preferred_element_type=jnp.float32)
                   + jnp.dot(jnp.cos(omd), wcd_ref[...],
                             preferred_element_type=jnp.float32)
                   + bd_ref[...]).astype(jnp.bfloat16)
    oma = (r * jnp.float32(A_STEP)) * div
    ta_ref[...] = (jnp.dot(jnp.sin(oma), wsa_ref[...],
                           preferred_element_type=jnp.float32)
                   + jnp.dot(jnp.cos(oma), wca_ref[...],
                             preferred_element_type=jnp.float32)
                   + ba_ref[...]).astype(jnp.bfloat16)


def _bins_body(pft_ref, pi_ref, bd_ref, ba_ref):
    pft = pft_ref[...]
    xj = pft[0:1, :]
    yj = pft[1:2, :]
    zj = pft[2:3, :]
    xi = pi_ref[:, 0:1]
    yi = pi_ref[:, 1:2]
    zi = pi_ref[:, 2:3]

    ax = xj - xi
    ay = yj - yi
    az = zj - zi
    s2 = ax * ax + ay * ay + az * az

    bd_ref[...] = s2.astype(jnp.int32)

    iota = jax.lax.broadcasted_iota(jnp.int32, (BI, N), 1)
    s2m = s2
    knn = []
    for t in range(ANGLE_K + 1):
        mn = jnp.min(s2m, axis=1, keepdims=True)
        idx = jnp.min(jnp.where(s2m == mn, iota, N), axis=1, keepdims=True)
        if t > 0:
            knn.append(idx)
        s2m = jnp.where(iota == idx, jnp.float32(jnp.inf), s2m)

    scale = jnp.float32(FACTOR_A * (1.0 / A_STEP))
    for t in range(ANGLE_K):
        onehot = (iota == knn[t]).astype(jnp.float32)
        rx = jnp.sum(onehot * xj, axis=1, keepdims=True) - xi
        ry = jnp.sum(onehot * yj, axis=1, keepdims=True) - yi
        rz = jnp.sum(onehot * zj, axis=1, keepdims=True) - zi
        cosv = rx * ax + ry * ay + rz * az
        cx = ry * az - rz * ay
        cy = rz * ax - rx * az
        cz = rx * ay - ry * ax
        sinv = jnp.sqrt(cx * cx + cy * cy + cz * cz)
        ang = jnp.arctan2(sinv, cosv)
        ba_ref[t] = jnp.round(ang * scale).astype(jnp.int32)


def _sc_body(td_hbm, ta_hbm, bd_hbm, ba0_hbm, ba1_hbm, ba2_hbm, out_hbm,
             idxd_v, idxa0_v, idxa1_v, idxa2_v,
             rd0, a00, a10, a20, ob0, rd1, a01, a11, a21, ob1, gs0, gs1):
    wid = lax.axis_index("s") * NC + lax.axis_index("c")
    base = wid * RPW
    pltpu.sync_copy(bd_hbm.at[pl.ds(base, RPW)], idxd_v)
    pltpu.sync_copy(ba0_hbm.at[pl.ds(base, RPW)], idxa0_v)
    pltpu.sync_copy(ba1_hbm.at[pl.ds(base, RPW)], idxa1_v)
    pltpu.sync_copy(ba2_hbm.at[pl.ds(base, RPW)], idxa2_v)

    bufs = ((rd0, a00, a10, a20, ob0, gs0), (rd1, a01, a11, a21, ob1, gs1))

    def fire(g, s):
        off = g * CH
        rd, a0, a1, a2, ob, gs = bufs[s]
        pltpu.async_copy(td_hbm.at[idxd_v.at[pl.ds(off, CH)]], rd, gs)
        pltpu.async_copy(ta_hbm.at[idxa0_v.at[pl.ds(off, CH)]], a0, gs)
        pltpu.async_copy(ta_hbm.at[idxa1_v.at[pl.ds(off, CH)]], a1, gs)
        pltpu.async_copy(ta_hbm.at[idxa2_v.at[pl.ds(off, CH)]], a2, gs)

    def drain(s):
        rd, a0, a1, a2, ob, gs = bufs[s]
        for dst in (rd, a0, a1, a2):
            pltpu.make_async_copy(td_hbm.at[pl.ds(0, CH)], dst, gs).wait()

    def compute_store(g, s):
        off = g * CH
        rd, a0, a1, a2, ob, gs = bufs[s]

        def unpk(w):
            return (jax.lax.bitcast_convert_type(w << 16, jnp.float32),
                    jax.lax.bitcast_convert_type(w, jnp.float32))

        @plsc.parallel_loop(0, CH, step=1, unroll=2)
        def rowbody(rr):
            for c in range(DIM // 32):
                sl = pl.ds(c * 16, 16)
                a0e, a0o = unpk(a0[rr, sl])
                a1e, a1o = unpk(a1[rr, sl])
                a2e, a2o = unpk(a2[rr, sl])
                t_ev, t_od = unpk(rd[rr, sl])
                m_ev = jnp.maximum(jnp.maximum(a0e, a1e), a2e)
                m_od = jnp.maximum(jnp.maximum(a0o, a1o), a2o)
                ob[rr, pl.ds(c * 32, 16)] = t_ev + m_ev
                ob[rr, pl.ds(c * 32 + 16, 16)] = t_od + m_od
        pltpu.sync_copy(ob, out_hbm.at[pl.ds(base + off, CH)])

    fire(0, 0)

    def outer(t, carry):
        g0 = t * 2
        fire(g0 + 1, 1)
        drain(0)
        compute_store(g0, 0)

        @pl.when(g0 + 2 < NCH)
        def _():
            fire(g0 + 2, 0)
        drain(1)
        compute_store(g0 + 1, 1)
        return carry

    lax.fori_loop(0, NCH // 2, outer, 0)


def _swizzle_perm():
    p = []
    for g in range(DIM // 32):
        for i in range(16):
            p.append(32 * g + i)
            p.append(32 * g + 16 + i)
    return p


_PERM = tuple(_swizzle_perm())


@jax.jit
def kernel(points, Wd, bd, Wa, ba):
    pft = points.reshape(N, 3).T.astype(jnp.float32)
    perm = jnp.asarray(_PERM, dtype=jnp.int32)
    WdT = Wd.T[:, perm]
    WaT = Wa.T[:, perm]
    bd = bd[perm]
    ba = ba[perm]
    wsd, wcd = WdT[0::2], WdT[1::2]
    wsa, wca = WaT[0::2], WaT[1::2]
    div = jnp.exp(jnp.arange(0, DIM, 2, dtype=jnp.float32)
                  * (-math.log(10000.0) / DIM)).reshape(1, DIM // 2)

    full = lambda i: (0, 0)
    wspec = pl.BlockSpec((DIM // 2, DIM), full)
    bspec = pl.BlockSpec((1, DIM), full)

    td, ta = pl.pallas_call(
        _tables_body,
        grid=(TROWS // RT,),
        in_specs=[wspec, wspec, bspec, wspec, wspec, bspec,
                  pl.BlockSpec((1, DIM // 2), full)],
        out_specs=[pl.BlockSpec((RT, DIM), lambda i: (i, 0)),
                   pl.BlockSpec((RT, DIM), lambda i: (i, 0))],
        out_shape=[jax.ShapeDtypeStruct((TROWS, DIM), jnp.bfloat16),
                   jax.ShapeDtypeStruct((TROWS, DIM), jnp.bfloat16)],
    )(wsd, wcd, bd.reshape(1, DIM), wsa, wca, ba.reshape(1, DIM), div)
    tdw = jax.lax.bitcast_convert_type(
        td.reshape(TROWS, DIM // 2, 2), jnp.int32)
    taw = jax.lax.bitcast_convert_type(
        ta.reshape(TROWS, DIM // 2, 2), jnp.int32)

    bd_idx, ba_idx = pl.pallas_call(
        _bins_body,
        grid=(N // BI,),
        in_specs=[pl.BlockSpec((3, N), full),
                  pl.BlockSpec((BI, 3), lambda i: (i, 0))],
        out_specs=[pl.BlockSpec((BI, N), lambda i: (i, 0)),
                   pl.BlockSpec((ANGLE_K, BI, N), lambda i: (0, i, 0))],
        out_shape=[jax.ShapeDtypeStruct((N, N), jnp.int32),
                   jax.ShapeDtypeStruct((ANGLE_K, N, N), jnp.int32)],
    )(pft, pft.T)

    sc = pl.kernel(
        _sc_body,
        out_type=jax.ShapeDtypeStruct((N * N, DIM), jnp.float32),
        mesh=plsc.VectorSubcoreMesh(core_axis_name="c", subcore_axis_name="s"),
        scratch_types=(
            [pltpu.VMEM((RPW,), jnp.int32)] * 4
            + ([pltpu.VMEM((CH, DIM // 2), jnp.int32)] * 4
               + [pltpu.VMEM((CH, DIM), jnp.float32)]) * 2
            + [pltpu.SemaphoreType.DMA, pltpu.SemaphoreType.DMA]
        ),
    )
    ba_flat = ba_idx.reshape(ANGLE_K, N * N)
    out = sc(tdw, taw, bd_idx.reshape(N * N),
             ba_flat[0], ba_flat[1], ba_flat[2])
    return out.reshape(1, N, N, DIM)


# speedup vs baseline: 1.4347x; 1.4115x over previous
import functools
import math

import jax
import jax.numpy as jnp
from jax import lax
from jax.experimental import pallas as pl
from jax.experimental.pallas import tpu as pltpu
from jax.experimental.pallas import tpu_sc as plsc

DIM = 256
N = 256
SIGMA_D = 0.2
SIGMA_A = 15.0
ANGLE_K = 3
FACTOR_A = 180.0 / (SIGMA_A * math.pi)

RT = 512
TROWS = 12288
BINS_A = 8192
A_STEP = 12.0 / (BINS_A - 1)
BI = 8

NC, NS = 2, 16
NW = NC * NS
RPW = (N * N) // NW
CH = 64
NCH = RPW // CH


def _pack_words(vals):
    lo = vals[:, :DIM // 2].astype(jnp.bfloat16).astype(jnp.float32)
    hi = vals[:, DIM // 2:].astype(jnp.bfloat16).astype(jnp.float32)
    lo_i = jax.lax.bitcast_convert_type(lo, jnp.int32)
    hi_i = jax.lax.bitcast_convert_type(hi, jnp.int32)
    return (hi_i & jnp.int32(-65536)) | jax.lax.shift_right_logical(lo_i, 16)


def _tables_body(wsd_ref, wcd_ref, bd_ref, wsa_ref, wca_ref, ba_ref,
                 div_ref, td_ref, ta_ref):
    base = pl.program_id(0) * RT
    r = (jax.lax.broadcasted_iota(jnp.int32, (RT, 1), 0)
         + base).astype(jnp.float32)
    div = div_ref[0:1, :]
    omd = (jnp.sqrt(r) / jnp.float32(SIGMA_D)) * div
    td_ref[...] = _pack_words(
        jnp.dot(jnp.sin(omd), wsd_ref[...],
                preferred_element_type=jnp.float32)
        + jnp.dot(jnp.cos(omd), wcd_ref[...],
                  preferred_element_type=jnp.float32)
        + bd_ref[...])
    oma = (r * jnp.float32(A_STEP)) * div
    ta_ref[...] = _pack_words(
        jnp.dot(jnp.sin(oma), wsa_ref[...],
                preferred_element_type=jnp.float32)
        + jnp.dot(jnp.cos(oma), wca_ref[...],
                  preferred_element_type=jnp.float32)
        + ba_ref[...])


def _bins_body(pft_ref, pi_ref, bd_ref, ba_ref):
    pft = pft_ref[...]
    xj = pft[0:1, :]
    yj = pft[1:2, :]
    zj = pft[2:3, :]
    xi = pi_ref[:, 0:1]
    yi = pi_ref[:, 1:2]
    zi = pi_ref[:, 2:3]

    ax = xj - xi
    ay = yj - yi
    az = zj - zi
    s2 = ax * ax + ay * ay + az * az

    bd_ref[...] = s2.astype(jnp.int32)

    iota = jax.lax.broadcasted_iota(jnp.int32, (BI, N), 1)
    s2m = s2
    knn = []
    for t in range(ANGLE_K + 1):
        mn = jnp.min(s2m, axis=1, keepdims=True)
        idx = jnp.min(jnp.where(s2m == mn, iota, N), axis=1, keepdims=True)
        if t > 0:
            knn.append(idx)
        s2m = jnp.where(iota == idx, jnp.float32(jnp.inf), s2m)

    scale = jnp.float32(FACTOR_A * (1.0 / A_STEP))
    for t in range(ANGLE_K):
        onehot = (iota == knn[t]).astype(jnp.float32)
        rx = jnp.sum(onehot * xj, axis=1, keepdims=True) - xi
        ry = jnp.sum(onehot * yj, axis=1, keepdims=True) - yi
        rz = jnp.sum(onehot * zj, axis=1, keepdims=True) - zi
        cosv = rx * ax + ry * ay + rz * az
        cx = ry * az - rz * ay
        cy = rz * ax - rx * az
        cz = rx * ay - ry * ax
        sinv = jnp.sqrt(cx * cx + cy * cy + cz * cz)
        ang = jnp.arctan2(sinv, cosv)
        ba_ref[t] = jnp.round(ang * scale).astype(jnp.int32)


def _sc_body(td_hbm, ta_hbm, bd_hbm, ba0_hbm, ba1_hbm, ba2_hbm, out_hbm,
             idxd_v, idxa0_v, idxa1_v, idxa2_v,
             rd0, a00, a10, a20, ob0, rd1, a01, a11, a21, ob1, gs0, gs1):
    wid = lax.axis_index("s") * NC + lax.axis_index("c")
    base = wid * RPW
    pltpu.sync_copy(bd_hbm.at[pl.ds(base, RPW)], idxd_v)
    pltpu.sync_copy(ba0_hbm.at[pl.ds(base, RPW)], idxa0_v)
    pltpu.sync_copy(ba1_hbm.at[pl.ds(base, RPW)], idxa1_v)
    pltpu.sync_copy(ba2_hbm.at[pl.ds(base, RPW)], idxa2_v)

    bufs = ((rd0, a00, a10, a20, ob0, gs0), (rd1, a01, a11, a21, ob1, gs1))

    def fire(g, s):
        off = g * CH
        rd, a0, a1, a2, ob, gs = bufs[s]
        pltpu.async_copy(td_hbm.at[idxd_v.at[pl.ds(off, CH)]], rd, gs)
        pltpu.async_copy(ta_hbm.at[idxa0_v.at[pl.ds(off, CH)]], a0, gs)
        pltpu.async_copy(ta_hbm.at[idxa1_v.at[pl.ds(off, CH)]], a1, gs)
        pltpu.async_copy(ta_hbm.at[idxa2_v.at[pl.ds(off, CH)]], a2, gs)

    def drain(s):
        rd, a0, a1, a2, ob, gs = bufs[s]
        for dst in (rd, a0, a1, a2):
            pltpu.make_async_copy(td_hbm.at[pl.ds(0, CH)], dst, gs).wait()

    def compute_store(g, s):
        off = g * CH
        rd, a0, a1, a2, ob, gs = bufs[s]

        def unpk(w):
            return (jax.lax.bitcast_convert_type(w << 16, jnp.float32),
                    jax.lax.bitcast_convert_type(w, jnp.float32))

        @plsc.parallel_loop(0, CH, step=1, unroll=2)
        def rowbody(rr):
            for c in range(DIM // 32):
                sl = pl.ds(c * 16, 16)
                a0e, a0o = unpk(a0[rr, sl])
                a1e, a1o = unpk(a1[rr, sl])
                a2e, a2o = unpk(a2[rr, sl])
                t_ev, t_od = unpk(rd[rr, sl])
                m_ev = jnp.maximum(jnp.maximum(a0e, a1e), a2e)
                m_od = jnp.maximum(jnp.maximum(a0o, a1o), a2o)
                ob[rr, pl.ds(c * 32, 16)] = t_ev + m_ev
                ob[rr, pl.ds(c * 32 + 16, 16)] = t_od + m_od
        pltpu.sync_copy(ob, out_hbm.at[pl.ds(base + off, CH)])

    fire(0, 0)

    def outer(t, carry):
        g0 = t * 2
        fire(g0 + 1, 1)
        drain(0)
        compute_store(g0, 0)

        @pl.when(g0 + 2 < NCH)
        def _():
            fire(g0 + 2, 0)
        drain(1)
        compute_store(g0 + 1, 1)
        return carry

    lax.fori_loop(0, NCH // 2, outer, 0)


def _swizzle_perm():
    lo = [32 * (k // 16) + k % 16 for k in range(DIM // 2)]
    hi = [32 * (k // 16) + 16 + k % 16 for k in range(DIM // 2)]
    return lo + hi


_PERM = tuple(_swizzle_perm())


@jax.jit
def kernel(points, Wd, bd, Wa, ba):
    pft = points.reshape(N, 3).T.astype(jnp.float32)
    perm = jnp.asarray(_PERM, dtype=jnp.int32)
    WdT = Wd.T[:, perm]
    WaT = Wa.T[:, perm]
    bd = bd[perm]
    ba = ba[perm]
    wsd, wcd = WdT[0::2], WdT[1::2]
    wsa, wca = WaT[0::2], WaT[1::2]
    div = jnp.exp(jnp.arange(0, DIM, 2, dtype=jnp.float32)
                  * (-math.log(10000.0) / DIM)).reshape(1, DIM // 2)

    full = lambda i: (0, 0)
    wspec = pl.BlockSpec((DIM // 2, DIM), full)
    bspec = pl.BlockSpec((1, DIM), full)

    td, ta = pl.pallas_call(
        _tables_body,
        grid=(TROWS // RT,),
        in_specs=[wspec, wspec, bspec, wspec, wspec, bspec,
                  pl.BlockSpec((1, DIM // 2), full)],
        out_specs=[pl.BlockSpec((RT, DIM // 2), lambda i: (i, 0)),
                   pl.BlockSpec((RT, DIM // 2), lambda i: (i, 0))],
        out_shape=[jax.ShapeDtypeStruct((TROWS, DIM // 2), jnp.int32),
                   jax.ShapeDtypeStruct((TROWS, DIM // 2), jnp.int32)],
    )(wsd, wcd, bd.reshape(1, DIM), wsa, wca, ba.reshape(1, DIM), div)
    tdw, taw = td, ta

    bd_idx, ba_idx = pl.pallas_call(
        _bins_body,
        grid=(N // BI,),
        in_specs=[pl.BlockSpec((3, N), full),
                  pl.BlockSpec((BI, 3), lambda i: (i, 0))],
        out_specs=[pl.BlockSpec((BI, N), lambda i: (i, 0)),
                   pl.BlockSpec((ANGLE_K, BI, N), lambda i: (0, i, 0))],
        out_shape=[jax.ShapeDtypeStruct((N, N), jnp.int32),
                   jax.ShapeDtypeStruct((ANGLE_K, N, N), jnp.int32)],
    )(pft, pft.T)

    sc = pl.kernel(
        _sc_body,
        out_type=jax.ShapeDtypeStruct((N * N, DIM), jnp.float32),
        mesh=plsc.VectorSubcoreMesh(core_axis_name="c", subcore_axis_name="s"),
        scratch_types=(
            [pltpu.VMEM((RPW,), jnp.int32)] * 4
            + ([pltpu.VMEM((CH, DIM // 2), jnp.int32)] * 4
               + [pltpu.VMEM((CH, DIM), jnp.float32)]) * 2
            + [pltpu.SemaphoreType.DMA, pltpu.SemaphoreType.DMA]
        ),
    )
    ba_flat = ba_idx.reshape(ANGLE_K, N * N)
    out = sc(tdw, taw, bd_idx.reshape(N * N),
             ba_flat[0], ba_flat[1], ba_flat[2])
    return out.reshape(1, N, N, DIM)


# speedup vs baseline: 1.4812x; 1.0324x over previous
import functools
import math

import jax
import jax.numpy as jnp
from jax import lax
from jax.experimental import pallas as pl
from jax.experimental.pallas import tpu as pltpu
from jax.experimental.pallas import tpu_sc as plsc

DIM = 256
N = 256
SIGMA_D = 0.2
SIGMA_A = 15.0
ANGLE_K = 3
FACTOR_A = 180.0 / (SIGMA_A * math.pi)

RT = 384
TROWS = 12288
BINS_A = 8192
A_STEP = 12.0 / (BINS_A - 1)
BI = 8

NC, NS = 2, 16
NW = NC * NS
RPW = (N * N) // NW
CH = 64
NCH = RPW // CH


def _pack_words(vals):
    lo = vals[:, :DIM // 2].astype(jnp.bfloat16).astype(jnp.float32)
    hi = vals[:, DIM // 2:].astype(jnp.bfloat16).astype(jnp.float32)
    lo_i = jax.lax.bitcast_convert_type(lo, jnp.int32)
    hi_i = jax.lax.bitcast_convert_type(hi, jnp.int32)
    return (hi_i & jnp.int32(-65536)) | jax.lax.shift_right_logical(lo_i, 16)


def _tc_body(pft_ref, pi_ref, wsd_ref, wcd_ref, bd_ref, wsa_ref, wca_ref,
             ba_ref, div_ref, td_ref, ta_ref, bdx_ref, bax_ref):
    base = pl.program_id(0) * RT
    r = (jax.lax.broadcasted_iota(jnp.int32, (RT, 1), 0)
         + base).astype(jnp.float32)
    div = div_ref[0:1, :]
    omd = (jnp.sqrt(r) / jnp.float32(SIGMA_D)) * div
    td_ref[...] = _pack_words(
        jnp.dot(jnp.sin(omd), wsd_ref[...],
                preferred_element_type=jnp.float32)
        + jnp.dot(jnp.cos(omd), wcd_ref[...],
                  preferred_element_type=jnp.float32)
        + bd_ref[...])
    oma = (r * jnp.float32(A_STEP)) * div
    ta_ref[...] = _pack_words(
        jnp.dot(jnp.sin(oma), wsa_ref[...],
                preferred_element_type=jnp.float32)
        + jnp.dot(jnp.cos(oma), wca_ref[...],
                  preferred_element_type=jnp.float32)
        + ba_ref[...])

    pft = pft_ref[...]
    xj = pft[0:1, :]
    yj = pft[1:2, :]
    zj = pft[2:3, :]
    xi = pi_ref[:, 0:1]
    yi = pi_ref[:, 1:2]
    zi = pi_ref[:, 2:3]

    ax = xj - xi
    ay = yj - yi
    az = zj - zi
    s2 = ax * ax + ay * ay + az * az

    bdx_ref[...] = s2.astype(jnp.int32)

    iota = jax.lax.broadcasted_iota(jnp.int32, (BI, N), 1)
    s2m = s2
    knn = []
    for t in range(ANGLE_K + 1):
        mn = jnp.min(s2m, axis=1, keepdims=True)
        idx = jnp.min(jnp.where(s2m == mn, iota, N), axis=1, keepdims=True)
        if t > 0:
            knn.append(idx)
        s2m = jnp.where(iota == idx, jnp.float32(jnp.inf), s2m)

    scale = jnp.float32(FACTOR_A * (1.0 / A_STEP))
    for t in range(ANGLE_K):
        onehot = (iota == knn[t]).astype(jnp.float32)
        rx = jnp.sum(onehot * xj, axis=1, keepdims=True) - xi
        ry = jnp.sum(onehot * yj, axis=1, keepdims=True) - yi
        rz = jnp.sum(onehot * zj, axis=1, keepdims=True) - zi
        cosv = rx * ax + ry * ay + rz * az
        cx = ry * az - rz * ay
        cy = rz * ax - rx * az
        cz = rx * ay - ry * ax
        sinv = jnp.sqrt(cx * cx + cy * cy + cz * cz)
        ang = jnp.arctan2(sinv, cosv)
        bax_ref[t] = jnp.round(ang * scale).astype(jnp.int32)


def _sc_body(td_hbm, ta_hbm, bd_hbm, ba0_hbm, ba1_hbm, ba2_hbm, out_hbm,
             idxd_v, idxa0_v, idxa1_v, idxa2_v,
             rd0, a00, a10, a20, ob0, rd1, a01, a11, a21, ob1, gs0, gs1):
    wid = lax.axis_index("s") * NC + lax.axis_index("c")
    base = wid * RPW
    pltpu.sync_copy(bd_hbm.at[pl.ds(base, RPW)], idxd_v)
    pltpu.sync_copy(ba0_hbm.at[pl.ds(base, RPW)], idxa0_v)
    pltpu.sync_copy(ba1_hbm.at[pl.ds(base, RPW)], idxa1_v)
    pltpu.sync_copy(ba2_hbm.at[pl.ds(base, RPW)], idxa2_v)

    bufs = ((rd0, a00, a10, a20, ob0, gs0), (rd1, a01, a11, a21, ob1, gs1))

    def fire(g, s):
        off = g * CH
        rd, a0, a1, a2, ob, gs = bufs[s]
        pltpu.async_copy(td_hbm.at[idxd_v.at[pl.ds(off, CH)]], rd, gs)
        pltpu.async_copy(ta_hbm.at[idxa0_v.at[pl.ds(off, CH)]], a0, gs)
        pltpu.async_copy(ta_hbm.at[idxa1_v.at[pl.ds(off, CH)]], a1, gs)
        pltpu.async_copy(ta_hbm.at[idxa2_v.at[pl.ds(off, CH)]], a2, gs)

    def drain(s):
        rd, a0, a1, a2, ob, gs = bufs[s]
        for dst in (rd, a0, a1, a2):
            pltpu.make_async_copy(td_hbm.at[pl.ds(0, CH)], dst, gs).wait()

    def compute_store(g, s):
        off = g * CH
        rd, a0, a1, a2, ob, gs = bufs[s]

        def unpk(w):
            return (jax.lax.bitcast_convert_type(w << 16, jnp.float32),
                    jax.lax.bitcast_convert_type(w, jnp.float32))

        @plsc.parallel_loop(0, CH, step=1, unroll=2)
        def rowbody(rr):
            for c in range(DIM // 32):
                sl = pl.ds(c * 16, 16)
                a0e, a0o = unpk(a0[rr, sl])
                a1e, a1o = unpk(a1[rr, sl])
                a2e, a2o = unpk(a2[rr, sl])
                t_ev, t_od = unpk(rd[rr, sl])
                m_ev = jnp.maximum(jnp.maximum(a0e, a1e), a2e)
                m_od = jnp.maximum(jnp.maximum(a0o, a1o), a2o)
                ob[rr, pl.ds(c * 32, 16)] = t_ev + m_ev
                ob[rr, pl.ds(c * 32 + 16, 16)] = t_od + m_od
        pltpu.sync_copy(ob, out_hbm.at[pl.ds(base + off, CH)])

    fire(0, 0)

    def outer(t, carry):
        g0 = t * 2
        fire(g0 + 1, 1)
        drain(0)
        compute_store(g0, 0)

        @pl.when(g0 + 2 < NCH)
        def _():
            fire(g0 + 2, 0)
        drain(1)
        compute_store(g0 + 1, 1)
        return carry

    lax.fori_loop(0, NCH // 2, outer, 0)


def _swizzle_perm():
    lo = [32 * (k // 16) + k % 16 for k in range(DIM // 2)]
    hi = [32 * (k // 16) + 16 + k % 16 for k in range(DIM // 2)]
    return lo + hi


_PERM = tuple(_swizzle_perm())


@jax.jit
def kernel(points, Wd, bd, Wa, ba):
    pft = points.reshape(N, 3).T.astype(jnp.float32)
    perm = jnp.asarray(_PERM, dtype=jnp.int32)
    WdT = Wd.T[:, perm]
    WaT = Wa.T[:, perm]
    bd = bd[perm]
    ba = ba[perm]
    wsd, wcd = WdT[0::2], WdT[1::2]
    wsa, wca = WaT[0::2], WaT[1::2]
    div = jnp.exp(jnp.arange(0, DIM, 2, dtype=jnp.float32)
                  * (-math.log(10000.0) / DIM)).reshape(1, DIM // 2)

    full = lambda i: (0, 0)
    wspec = pl.BlockSpec((DIM // 2, DIM), full)
    bspec = pl.BlockSpec((1, DIM), full)

    tdw, taw, bd_idx, ba_idx = pl.pallas_call(
        _tc_body,
        grid=(N // BI,),
        in_specs=[pl.BlockSpec((3, N), full),
                  pl.BlockSpec((BI, 3), lambda i: (i, 0)),
                  wspec, wspec, bspec, wspec, wspec, bspec,
                  pl.BlockSpec((1, DIM // 2), full)],
        out_specs=[pl.BlockSpec((RT, DIM // 2), lambda i: (i, 0)),
                   pl.BlockSpec((RT, DIM // 2), lambda i: (i, 0)),
                   pl.BlockSpec((BI, N), lambda i: (i, 0)),
                   pl.BlockSpec((ANGLE_K, BI, N), lambda i: (0, i, 0))],
        out_shape=[jax.ShapeDtypeStruct((TROWS, DIM // 2), jnp.int32),
                   jax.ShapeDtypeStruct((TROWS, DIM // 2), jnp.int32),
                   jax.ShapeDtypeStruct((N, N), jnp.int32),
                   jax.ShapeDtypeStruct((ANGLE_K, N, N), jnp.int32)],
    )(pft, pft.T, wsd, wcd, bd.reshape(1, DIM), wsa, wca, ba.reshape(1, DIM),
      div)

    sc = pl.kernel(
        _sc_body,
        out_type=jax.ShapeDtypeStruct((N * N, DIM), jnp.float32),
        mesh=plsc.VectorSubcoreMesh(core_axis_name="c", subcore_axis_name="s"),
        scratch_types=(
            [pltpu.VMEM((RPW,), jnp.int32)] * 4
            + ([pltpu.VMEM((CH, DIM // 2), jnp.int32)] * 4
               + [pltpu.VMEM((CH, DIM), jnp.float32)]) * 2
            + [pltpu.SemaphoreType.DMA, pltpu.SemaphoreType.DMA]
        ),
    )
    ba_flat = ba_idx.reshape(ANGLE_K, N * N)
    out = sc(tdw, taw, bd_idx.reshape(N * N),
             ba_flat[0], ba_flat[1], ba_flat[2])
    return out.reshape(1, N, N, DIM)
